# Initial kernel scaffold; baseline (speedup 1.0000x reference)
import jax
import jax.numpy as jnp
from jax.experimental import pallas as pl


def kernel(feat, sup_indices, sup_values, W0, b0, Wg0, Wg1, We, be, Wl0, bl0, Wl1, bl1, Wr0, br0, Wr1, br1):
    raise NotImplementedError("write your pallas kernel here")


# speedup vs baseline: 3.2631x; 3.2631x over previous
import functools

import jax
import jax.numpy as jnp
from jax import lax
from jax.experimental import pallas as pl
from jax.experimental.pallas import tpu as pltpu
from jax.experimental.pallas import tpu_sc as plsc

N = 50000
E = 800000
D_IN = 512
D = 64
H = 32

NC = 2
NS = 16
EPT = E // NS
SUB = 80
SUBN = 5
CH = SUB * SUBN
NCHUNK = EPT // CH
RPT = N // NS
ZR = 625


def _l2norm(x):
    n = jnp.sqrt(jnp.sum(x * x, axis=1, keepdims=True))
    return x / jnp.maximum(n, 1e-12)


BN = 2000


def _mlp0_body(feat_ref, w0_ref, b0_ref, hs_ref, h_ref):
    h = jnp.tanh(
        jnp.dot(feat_ref[...], w0_ref[...], preferred_element_type=jnp.float32)
        + b0_ref[...]
    )
    h_ref[...] = h
    hs_ref[0] = h[:, :H]
    hs_ref[1] = h[:, H:]


def _mlp0(feat, W0, b0):
    grid = (N // BN,)
    return pl.pallas_call(
        _mlp0_body,
        grid=grid,
        in_specs=[
            pl.BlockSpec((BN, D_IN), lambda i: (i, 0)),
            pl.BlockSpec((D_IN, D), lambda i: (0, 0)),
            pl.BlockSpec((1, D), lambda i: (0, 0)),
        ],
        out_specs=[
            pl.BlockSpec((NC, BN, H), lambda i: (0, i, 0)),
            pl.BlockSpec((BN, D), lambda i: (i, 0)),
        ],
        out_shape=[
            jax.ShapeDtypeStruct((NC, N, H), jnp.float32),
            jax.ShapeDtypeStruct((N, D), jnp.float32),
        ],
    )(feat, W0, b0.reshape(1, D))


def _spmm_body(tbl_hbm, rows_hbm, cols_hbm, vals_hbm, out_hbm,
               rows_buf, cols_i, dst_i, vals_v, table, sem):
    c = lax.axis_index("c")
    s = lax.axis_index("s")

    def zb(r, carry):
        rows_buf[r, pl.ds(0, 16)] = jnp.zeros((16,), jnp.float32)
        rows_buf[r, pl.ds(16, 16)] = jnp.zeros((16,), jnp.float32)
        return carry

    lax.fori_loop(0, CH, zb, 0)
    for i in range(RPT // CH):
        pltpu.sync_copy(rows_buf, table.at[pl.ds(s * RPT + i * CH, CH)])
    rem = RPT - (RPT // CH) * CH
    if rem:
        pltpu.sync_copy(rows_buf.at[pl.ds(0, rem)],
                        table.at[pl.ds(s * RPT + (RPT // CH) * CH, rem)])
    plsc.subcore_barrier()

    my_tbl = tbl_hbm.at[c]

    def chunk(j, carry):
        base = s * EPT + j * CH
        for i in range(SUBN):
            pltpu.sync_copy(cols_hbm.at[pl.ds(base + i * SUB, SUB)],
                            cols_i.at[i])
            pltpu.sync_copy(rows_hbm.at[pl.ds(base + i * SUB, SUB)],
                            dst_i.at[i])
        pltpu.sync_copy(vals_hbm.at[pl.ds(base, CH)], vals_v)
        descs = [
            pltpu.async_copy(my_tbl.at[cols_i.at[i]],
                             rows_buf.at[pl.ds(i * SUB, SUB)], sem)
            for i in range(SUBN)
        ]
        for d in descs:
            d.wait()

        def scale(g, carry2):
            v16 = vals_v[pl.ds(g * 16, 16)]
            e0 = g * 16
            for i in range(16):
                v = v16[i]
                rows_buf[e0 + i, pl.ds(0, 16)] = rows_buf[e0 + i, pl.ds(0, 16)] * v
                rows_buf[e0 + i, pl.ds(16, 16)] = rows_buf[e0 + i, pl.ds(16, 16)] * v
            return carry2

        lax.fori_loop(0, CH // 16, scale, 0)
        for i in range(SUBN):
            pltpu.sync_copy(rows_buf.at[pl.ds(i * SUB, SUB)],
                            table.at[dst_i.at[i]], add=True)
        return carry

    lax.fori_loop(0, NCHUNK, chunk, 0)
    plsc.subcore_barrier()
    pltpu.sync_copy(table.at[pl.ds(s * RPT, RPT)],
                    out_hbm.at[c].at[pl.ds(s * RPT, RPT)])


def _spmm_sc(tbl, rows, cols, vals):
    mesh = plsc.VectorSubcoreMesh(core_axis_name="c", subcore_axis_name="s",
                                  num_cores=NC, num_subcores=NS)
    fn = functools.partial(
        pl.kernel,
        out_type=jax.ShapeDtypeStruct((NC, N, H), jnp.float32),
        mesh=mesh,
        scratch_types=[
            pltpu.VMEM((CH, H), jnp.float32),
            pltpu.VMEM((SUBN, SUB), jnp.int32),
            pltpu.VMEM((SUBN, SUB), jnp.int32),
            pltpu.VMEM((CH,), jnp.float32),
            pltpu.VMEM_SHARED((N, H), jnp.float32),
            pltpu.SemaphoreType.DMA,
        ],
        compiler_params=pltpu.CompilerParams(use_tc_tiling_on_sc=False),
    )(_spmm_body)
    return fn(tbl, rows, cols, vals)


def _layer_body(ss_ref, wg_ref, accin_ref, gs_ref, acc_ref):
    t = (jnp.dot(ss_ref[0], wg_ref[:H, :], preferred_element_type=jnp.float32)
         + jnp.dot(ss_ref[1], wg_ref[H:, :], preferred_element_type=jnp.float32))
    g = _l2norm(jnp.tanh(t))
    acc_ref[...] = accin_ref[...] + g
    gs_ref[0] = g[:, :H]
    gs_ref[1] = g[:, H:]


def _layer(ss, Wg, accin):
    grid = (N // BN,)
    return pl.pallas_call(
        _layer_body,
        grid=grid,
        in_specs=[
            pl.BlockSpec((NC, BN, H), lambda i: (0, i, 0)),
            pl.BlockSpec((D, D), lambda i: (0, 0)),
            pl.BlockSpec((BN, D), lambda i: (i, 0)),
        ],
        out_specs=[
            pl.BlockSpec((NC, BN, H), lambda i: (0, i, 0)),
            pl.BlockSpec((BN, D), lambda i: (i, 0)),
        ],
        out_shape=[
            jax.ShapeDtypeStruct((NC, N, H), jnp.float32),
            jax.ShapeDtypeStruct((N, D), jnp.float32),
        ],
    )(ss, Wg, accin)


def _final_body(ss_ref, wg_ref, accin_ref, we_ref, be_ref,
                wl0_ref, bl0_ref, wl1_ref, bl1_ref,
                wr0_ref, br0_ref, wr1_ref, br1_ref,
                emb_ref, lft_ref, rgt_ref):
    t = (jnp.dot(ss_ref[0], wg_ref[:H, :], preferred_element_type=jnp.float32)
         + jnp.dot(ss_ref[1], wg_ref[H:, :], preferred_element_type=jnp.float32))
    g = _l2norm(jnp.tanh(t))
    acc = accin_ref[...] + g
    emb_ref[...] = _l2norm(
        jnp.dot(acc, we_ref[...], preferred_element_type=jnp.float32)
        + be_ref[...])
    lo = jnp.tanh(jnp.dot(g, wl0_ref[...], preferred_element_type=jnp.float32)
                  + bl0_ref[...]) + g
    lft_ref[...] = jnp.tanh(
        jnp.dot(lo, wl1_ref[...], preferred_element_type=jnp.float32)
        + bl1_ref[...])
    ro = jnp.tanh(jnp.dot(g, wr0_ref[...], preferred_element_type=jnp.float32)
                  + br0_ref[...]) + g
    rgt_ref[...] = jnp.tanh(
        jnp.dot(ro, wr1_ref[...], preferred_element_type=jnp.float32)
        + br1_ref[...])


def _final(ss, Wg, accin, We, be, Wl0, bl0, Wl1, bl1, Wr0, br0, Wr1, br1):
    grid = (N // BN,)
    mat = pl.BlockSpec((D, D), lambda i: (0, 0))
    vec = pl.BlockSpec((1, D), lambda i: (0, 0))
    blk = pl.BlockSpec((BN, D), lambda i: (i, 0))
    return pl.pallas_call(
        _final_body,
        grid=grid,
        in_specs=[
            pl.BlockSpec((NC, BN, H), lambda i: (0, i, 0)),
            mat, blk, mat, vec, mat, vec, mat, vec, mat, vec, mat, vec,
        ],
        out_specs=[blk, blk, blk],
        out_shape=[
            jax.ShapeDtypeStruct((N, D), jnp.float32),
            jax.ShapeDtypeStruct((N, D), jnp.float32),
            jax.ShapeDtypeStruct((N, D), jnp.float32),
        ],
    )(ss, Wg, accin, We, be.reshape(1, D),
      Wl0, bl0.reshape(1, D), Wl1, bl1.reshape(1, D),
      Wr0, br0.reshape(1, D), Wr1, br1.reshape(1, D))


def kernel(feat, sup_indices, sup_values, W0, b0, Wg0, Wg1, We, be,
           Wl0, bl0, Wl1, bl1, Wr0, br0, Wr1, br1):
    rows = sup_indices[0]
    cols = sup_indices[1]
    hs, h = _mlp0(feat, W0, b0)
    s1 = _spmm_sc(hs, rows, cols, sup_values)
    gs1, acc1 = _layer(s1, Wg0, h)
    s2 = _spmm_sc(gs1, rows, cols, sup_values)
    emb, lft, rgt = _final(s2, Wg1, acc1, We, be,
                           Wl0, bl0, Wl1, bl1, Wr0, br0, Wr1, br1)
    return (emb, lft, rgt)


# speedup vs baseline: 8.6240x; 2.6429x over previous
import functools

import jax
import jax.numpy as jnp
from jax import lax
from jax.experimental import pallas as pl
from jax.experimental.pallas import tpu as pltpu
from jax.experimental.pallas import tpu_sc as plsc

N = 50000
E = 800000
D_IN = 512
D = 64
H = 32

NC = 2
NS = 16
EPT = E // NS
SUB = 400
SUBN = 1
CH = SUB * SUBN
NCHUNK = EPT // CH
RPT = N // NS
ZR = 625


def _l2norm(x):
    n = jnp.sqrt(jnp.sum(x * x, axis=1, keepdims=True))
    return x / jnp.maximum(n, 1e-12)


BN = 2000


def _mlp0_body(feat_ref, w0_ref, b0_ref, hs_ref, h_ref):
    h = jnp.tanh(
        jnp.dot(feat_ref[...], w0_ref[...], preferred_element_type=jnp.float32)
        + b0_ref[...]
    )
    h_ref[...] = h
    hs_ref[0] = h[:, :H]
    hs_ref[1] = h[:, H:]


def _mlp0(feat, W0, b0):
    grid = (N // BN,)
    return pl.pallas_call(
        _mlp0_body,
        grid=grid,
        in_specs=[
            pl.BlockSpec((BN, D_IN), lambda i: (i, 0)),
            pl.BlockSpec((D_IN, D), lambda i: (0, 0)),
            pl.BlockSpec((1, D), lambda i: (0, 0)),
        ],
        out_specs=[
            pl.BlockSpec((NC, BN, H), lambda i: (0, i, 0)),
            pl.BlockSpec((BN, D), lambda i: (i, 0)),
        ],
        out_shape=[
            jax.ShapeDtypeStruct((NC, N, H), jnp.float32),
            jax.ShapeDtypeStruct((N, D), jnp.float32),
        ],
    )(feat, W0, b0.reshape(1, D))


def _spmm_body(tbl_hbm, edges_hbm, out_hbm, rows_buf, idx_b, table, *sems):
    c = lax.axis_index("c")
    s = lax.axis_index("s")
    gsem = sems[0:2]
    ssem = sems[2:4]
    isem = sems[4:8]

    def zb(r, carry):
        rows_buf[0, r, pl.ds(0, 16)] = jnp.zeros((16,), jnp.float32)
        rows_buf[0, r, pl.ds(16, 16)] = jnp.zeros((16,), jnp.float32)
        return carry

    lax.fori_loop(0, CH, zb, 0)
    for i in range(RPT // CH):
        pltpu.sync_copy(rows_buf.at[0], table.at[pl.ds(s * RPT + i * CH, CH)])
    rem = RPT - (RPT // CH) * CH
    if rem:
        pltpu.sync_copy(rows_buf.at[0].at[pl.ds(0, rem)],
                        table.at[pl.ds(s * RPT + (RPT // CH) * CH, rem)])
    plsc.subcore_barrier()

    my_tbl = tbl_hbm.at[c]
    cid0 = s * NCHUNK

    def issue_idx(j, q):
        pltpu.async_copy(edges_hbm.at[cid0 + j], idx_b.at[q], isem[q])

    def wait_idx(j, q):
        pltpu.make_async_copy(edges_hbm.at[cid0 + j], idx_b.at[q],
                              isem[q]).wait()

    def issue_gather(b, q):
        pltpu.async_copy(my_tbl.at[idx_b.at[q].at[0]], rows_buf.at[b],
                         gsem[b])

    def wait_gather(b, q):
        pltpu.make_async_copy(my_tbl.at[idx_b.at[q].at[0]], rows_buf.at[b],
                              gsem[b]).wait()

    def issue_scatter(b, q):
        pltpu.async_copy(rows_buf.at[b], table.at[idx_b.at[q].at[1]],
                         ssem[b], add=True)

    def wait_scatter(b, q):
        pltpu.make_async_copy(rows_buf.at[b], table.at[idx_b.at[q].at[1]],
                              ssem[b]).wait()

    def scale(b, q):
        def grp(g, carry):
            v16 = plsc.bitcast(idx_b[q, 2, pl.ds(g * 16, 16)], jnp.float32)
            e0 = g * 16
            for i in range(16):
                v = v16[i]
                rows_buf[b, e0 + i, pl.ds(0, 16)] = \
                    rows_buf[b, e0 + i, pl.ds(0, 16)] * v
                rows_buf[b, e0 + i, pl.ds(16, 16)] = \
                    rows_buf[b, e0 + i, pl.ds(16, 16)] * v
            return carry

        lax.fori_loop(0, CH // 16, grp, 0)

    def chunk_steady(j, u, g_next=True, idx_next=True):
        b = (1 + u) % 2
        q = (1 + u) % 4
        wait_scatter(1 - b, u % 4)
        if g_next:
            wait_idx(j + 1, (2 + u) % 4)
            issue_gather(1 - b, (2 + u) % 4)
        if idx_next:
            issue_idx(j + 3, u % 4)
        wait_gather(b, q)
        scale(b, q)
        issue_scatter(b, q)

    issue_idx(0, 0)
    issue_idx(1, 1)
    issue_idx(2, 2)
    wait_idx(0, 0)
    issue_gather(0, 0)
    wait_idx(1, 1)
    issue_gather(1, 1)
    issue_idx(3, 3)
    wait_gather(0, 0)
    scale(0, 0)
    issue_scatter(0, 0)

    def steady4(g, carry):
        jb = 1 + 4 * g
        for u in range(4):
            chunk_steady(jb + u, u)
        return carry

    lax.fori_loop(0, (NCHUNK - 5) // 4, steady4, 0)

    for u in range(4):
        j = NCHUNK - 4 + u
        chunk_steady(j, u, g_next=(j + 1 <= NCHUNK - 1),
                     idx_next=(j + 3 <= NCHUNK - 1))
    wait_scatter(0, 0)

    plsc.subcore_barrier()
    pltpu.sync_copy(table.at[pl.ds(s * RPT, RPT)],
                    out_hbm.at[c].at[pl.ds(s * RPT, RPT)])


def _spmm_sc(tbl, edges):
    mesh = plsc.VectorSubcoreMesh(core_axis_name="c", subcore_axis_name="s",
                                  num_cores=NC, num_subcores=NS)
    fn = functools.partial(
        pl.kernel,
        out_type=jax.ShapeDtypeStruct((NC, N, H), jnp.float32),
        mesh=mesh,
        scratch_types=[
            pltpu.VMEM((2, CH, H), jnp.float32),
            pltpu.VMEM((4, 3, CH), jnp.int32),
            pltpu.VMEM_SHARED((N, H), jnp.float32),
            pltpu.SemaphoreType.DMA,
            pltpu.SemaphoreType.DMA,
            pltpu.SemaphoreType.DMA,
            pltpu.SemaphoreType.DMA,
            pltpu.SemaphoreType.DMA,
            pltpu.SemaphoreType.DMA,
            pltpu.SemaphoreType.DMA,
            pltpu.SemaphoreType.DMA,
        ],
        compiler_params=pltpu.CompilerParams(use_tc_tiling_on_sc=False, needs_layout_passes=False),
    )(_spmm_body)
    return fn(tbl, edges)


def _layer_body(ss_ref, wg_ref, accin_ref, gs_ref, acc_ref):
    t = (jnp.dot(ss_ref[0], wg_ref[:H, :], preferred_element_type=jnp.float32)
         + jnp.dot(ss_ref[1], wg_ref[H:, :], preferred_element_type=jnp.float32))
    g = _l2norm(jnp.tanh(t))
    acc_ref[...] = accin_ref[...] + g
    gs_ref[0] = g[:, :H]
    gs_ref[1] = g[:, H:]


def _layer(ss, Wg, accin):
    grid = (N // BN,)
    return pl.pallas_call(
        _layer_body,
        grid=grid,
        in_specs=[
            pl.BlockSpec((NC, BN, H), lambda i: (0, i, 0)),
            pl.BlockSpec((D, D), lambda i: (0, 0)),
            pl.BlockSpec((BN, D), lambda i: (i, 0)),
        ],
        out_specs=[
            pl.BlockSpec((NC, BN, H), lambda i: (0, i, 0)),
            pl.BlockSpec((BN, D), lambda i: (i, 0)),
        ],
        out_shape=[
            jax.ShapeDtypeStruct((NC, N, H), jnp.float32),
            jax.ShapeDtypeStruct((N, D), jnp.float32),
        ],
    )(ss, Wg, accin)


def _final_body(ss_ref, wg_ref, accin_ref, we_ref, be_ref,
                wl0_ref, bl0_ref, wl1_ref, bl1_ref,
                wr0_ref, br0_ref, wr1_ref, br1_ref,
                emb_ref, lft_ref, rgt_ref):
    t = (jnp.dot(ss_ref[0], wg_ref[:H, :], preferred_element_type=jnp.float32)
         + jnp.dot(ss_ref[1], wg_ref[H:, :], preferred_element_type=jnp.float32))
    g = _l2norm(jnp.tanh(t))
    acc = accin_ref[...] + g
    emb_ref[...] = _l2norm(
        jnp.dot(acc, we_ref[...], preferred_element_type=jnp.float32)
        + be_ref[...])
    lo = jnp.tanh(jnp.dot(g, wl0_ref[...], preferred_element_type=jnp.float32)
                  + bl0_ref[...]) + g
    lft_ref[...] = jnp.tanh(
        jnp.dot(lo, wl1_ref[...], preferred_element_type=jnp.float32)
        + bl1_ref[...])
    ro = jnp.tanh(jnp.dot(g, wr0_ref[...], preferred_element_type=jnp.float32)
                  + br0_ref[...]) + g
    rgt_ref[...] = jnp.tanh(
        jnp.dot(ro, wr1_ref[...], preferred_element_type=jnp.float32)
        + br1_ref[...])


def _final(ss, Wg, accin, We, be, Wl0, bl0, Wl1, bl1, Wr0, br0, Wr1, br1):
    grid = (N // BN,)
    mat = pl.BlockSpec((D, D), lambda i: (0, 0))
    vec = pl.BlockSpec((1, D), lambda i: (0, 0))
    blk = pl.BlockSpec((BN, D), lambda i: (i, 0))
    return pl.pallas_call(
        _final_body,
        grid=grid,
        in_specs=[
            pl.BlockSpec((NC, BN, H), lambda i: (0, i, 0)),
            mat, blk, mat, vec, mat, vec, mat, vec, mat, vec, mat, vec,
        ],
        out_specs=[blk, blk, blk],
        out_shape=[
            jax.ShapeDtypeStruct((N, D), jnp.float32),
            jax.ShapeDtypeStruct((N, D), jnp.float32),
            jax.ShapeDtypeStruct((N, D), jnp.float32),
        ],
    )(ss, Wg, accin, We, be.reshape(1, D),
      Wl0, bl0.reshape(1, D), Wl1, bl1.reshape(1, D),
      Wr0, br0.reshape(1, D), Wr1, br1.reshape(1, D))


def kernel(feat, sup_indices, sup_values, W0, b0, Wg0, Wg1, We, be,
           Wl0, bl0, Wl1, bl1, Wr0, br0, Wr1, br1):
    rows = sup_indices[0]
    cols = sup_indices[1]
    edges = jnp.stack(
        [cols.reshape(E // CH, CH).astype(jnp.int32),
         rows.reshape(E // CH, CH).astype(jnp.int32),
         lax.bitcast_convert_type(sup_values, jnp.int32).reshape(E // CH, CH)],
        axis=1)
    hs, h = _mlp0(feat, W0, b0)
    s1 = _spmm_sc(hs, edges)
    gs1, acc1 = _layer(s1, Wg0, h)
    s2 = _spmm_sc(gs1, edges)
    emb, lft, rgt = _final(s2, Wg1, acc1, We, be,
                           Wl0, bl0, Wl1, bl1, Wr0, br0, Wr1, br1)
    return (emb, lft, rgt)


# speedup vs baseline: 10.0743x; 1.1682x over previous
import functools

import jax
import jax.numpy as jnp
from jax import lax
from jax.experimental import pallas as pl
from jax.experimental.pallas import tpu as pltpu
from jax.experimental.pallas import tpu_sc as plsc

N = 50000
E = 800000
D_IN = 512
D = 64
H = 32
NP = N // 2

NC = 2
NS = 16
EPT = E // NS
CH = 400
NCHUNK = EPT // CH
RPT = N // NS
BN = 2000
BNP = BN // 2


def _l2norm_p(x, bsum):
    n2 = jnp.dot(x * x, bsum, preferred_element_type=jnp.float32)
    return x * lax.rsqrt(jnp.maximum(n2, 1e-24))


def _mlp0_body(feat_ref, w0_ref, b0_ref, hp_ref):
    hp_ref[...] = jnp.tanh(
        jnp.dot(feat_ref[...], w0_ref[...], preferred_element_type=jnp.float32)
        + b0_ref[...]
    )


def _mlp0(featp, W02, b02):
    return pl.pallas_call(
        _mlp0_body,
        grid=(NP // BNP,),
        in_specs=[
            pl.BlockSpec((BNP, 2 * D_IN), lambda i: (i, 0)),
            pl.BlockSpec((2 * D_IN, 128), lambda i: (0, 0)),
            pl.BlockSpec((1, 128), lambda i: (0, 0)),
        ],
        out_specs=pl.BlockSpec((BNP, 128), lambda i: (i, 0)),
        out_shape=jax.ShapeDtypeStruct((NP, 128), jnp.float32),
    )(featp, W02, b02)


def _spmm_body(tbl_hbm, dst_hbm, col_hbm, val_hbm, out_hbm,
               rows_buf, col_b, dst_b, val_b, table, *sems):
    c = lax.axis_index("c")
    s = lax.axis_index("s")
    gsem = sems[0:2]
    ssem = sems[2:4]
    isem = sems[4:8]

    def zb(r, carry):
        rows_buf[0, r, pl.ds(0, 16)] = jnp.zeros((16,), jnp.float32)
        rows_buf[0, r, pl.ds(16, 16)] = jnp.zeros((16,), jnp.float32)
        return carry

    lax.fori_loop(0, CH, zb, 0)
    for i in range(RPT // CH):
        pltpu.sync_copy(rows_buf.at[0], table.at[pl.ds(s * RPT + i * CH, CH)])
    rem = RPT - (RPT // CH) * CH
    if rem:
        pltpu.sync_copy(rows_buf.at[0].at[pl.ds(0, rem)],
                        table.at[pl.ds(s * RPT + (RPT // CH) * CH, rem)])
    plsc.subcore_barrier()

    base0 = s * EPT

    def issue_idx(j, q):
        b = base0 + j * CH
        pltpu.async_copy(col_hbm.at[pl.ds(b, CH)], col_b.at[q], isem[q])
        pltpu.async_copy(dst_hbm.at[pl.ds(b, CH)], dst_b.at[q], isem[q])
        pltpu.async_copy(val_hbm.at[pl.ds(b, CH)], val_b.at[q], isem[q])

    def wait_idx(j, q):
        b = base0 + j * CH
        pltpu.make_async_copy(col_hbm.at[pl.ds(b, CH)], col_b.at[q],
                              isem[q]).wait()
        pltpu.make_async_copy(dst_hbm.at[pl.ds(b, CH)], dst_b.at[q],
                              isem[q]).wait()
        pltpu.make_async_copy(val_hbm.at[pl.ds(b, CH)], val_b.at[q],
                              isem[q]).wait()

        def xf(g, carry):
            v = col_b[q, pl.ds(g * 16, 16)]
            col_b[q, pl.ds(g * 16, 16)] = v * 2 + c
            return carry

        lax.fori_loop(0, CH // 16, xf, 0)

    def issue_gather(b, q):
        pltpu.async_copy(tbl_hbm.at[col_b.at[q]], rows_buf.at[b], gsem[b])

    def wait_gather(b, q):
        pltpu.make_async_copy(tbl_hbm.at[col_b.at[q]], rows_buf.at[b],
                              gsem[b]).wait()

    def issue_scatter(b, q):
        pltpu.async_copy(rows_buf.at[b], table.at[dst_b.at[q]],
                         ssem[b], add=True)

    def wait_scatter(b, q):
        pltpu.make_async_copy(rows_buf.at[b], table.at[dst_b.at[q]],
                              ssem[b]).wait()

    def scale(b, q):
        def grp(g, carry):
            v16 = val_b[q, pl.ds(g * 16, 16)]
            e0 = g * 16
            for i in range(16):
                v = v16[i]
                rows_buf[b, e0 + i, pl.ds(0, 16)] = \
                    rows_buf[b, e0 + i, pl.ds(0, 16)] * v
                rows_buf[b, e0 + i, pl.ds(16, 16)] = \
                    rows_buf[b, e0 + i, pl.ds(16, 16)] * v
            return carry

        lax.fori_loop(0, CH // 16, grp, 0)

    def chunk_steady(j, u, g_next=True, idx_next=True):
        b = (1 + u) % 2
        q = (1 + u) % 4
        wait_scatter(1 - b, u % 4)
        if g_next:
            wait_idx(j + 1, (2 + u) % 4)
            issue_gather(1 - b, (2 + u) % 4)
        if idx_next:
            issue_idx(j + 3, u % 4)
        wait_gather(b, q)
        scale(b, q)
        issue_scatter(b, q)

    issue_idx(0, 0)
    issue_idx(1, 1)
    issue_idx(2, 2)
    wait_idx(0, 0)
    issue_gather(0, 0)
    wait_idx(1, 1)
    issue_gather(1, 1)
    issue_idx(3, 3)
    wait_gather(0, 0)
    scale(0, 0)
    issue_scatter(0, 0)

    def steady4(g, carry):
        jb = 1 + 4 * g
        for u in range(4):
            chunk_steady(jb + u, u)
        return carry

    lax.fori_loop(0, (NCHUNK - 5) // 4, steady4, 0)

    for u in range(4):
        j = NCHUNK - 4 + u
        chunk_steady(j, u, g_next=(j + 1 <= NCHUNK - 1),
                     idx_next=(j + 3 <= NCHUNK - 1))
    wait_scatter(0, 0)

    plsc.subcore_barrier()
    lo = s * RPT
    for k in range(NC):
        @pl.when(c == k)
        def _():
            pltpu.sync_copy(table.at[pl.ds(lo, RPT)],
                            out_hbm.at[pl.ds(lo, RPT), k])


def _spmm_sc(hp, dst, col, val):
    tbl = jnp.reshape(hp, (2 * N, H))
    mesh = plsc.VectorSubcoreMesh(core_axis_name="c", subcore_axis_name="s",
                                  num_cores=NC, num_subcores=NS)
    fn = functools.partial(
        pl.kernel,
        out_type=jax.ShapeDtypeStruct((N, NC, H), jnp.float32),
        mesh=mesh,
        scratch_types=[
            pltpu.VMEM((2, CH, H), jnp.float32),
            pltpu.VMEM((4, CH), jnp.int32),
            pltpu.VMEM((4, CH), jnp.int32),
            pltpu.VMEM((4, CH), jnp.float32),
            pltpu.VMEM_SHARED((N, H), jnp.float32),
            pltpu.SemaphoreType.DMA,
            pltpu.SemaphoreType.DMA,
            pltpu.SemaphoreType.DMA,
            pltpu.SemaphoreType.DMA,
            pltpu.SemaphoreType.DMA,
            pltpu.SemaphoreType.DMA,
            pltpu.SemaphoreType.DMA,
            pltpu.SemaphoreType.DMA,
        ],
        compiler_params=pltpu.CompilerParams(use_tc_tiling_on_sc=False,
                                             needs_layout_passes=False),
    )(_spmm_body)
    out = fn(tbl, dst, col, val)
    return jnp.reshape(out, (NP, 128))


def _layer_body(sp_ref, wg_ref, accin_ref, bsum_ref, gp_ref, accp_ref):
    g = _l2norm_p(jnp.tanh(
        jnp.dot(sp_ref[...], wg_ref[...], preferred_element_type=jnp.float32)),
        bsum_ref[...])
    accp_ref[...] = accin_ref[...] + g
    gp_ref[...] = g


def _layer(sp, Wg2, accin_p, bsum):
    blk = pl.BlockSpec((BNP, 128), lambda i: (i, 0))
    mat = pl.BlockSpec((128, 128), lambda i: (0, 0))
    return pl.pallas_call(
        _layer_body,
        grid=(NP // BNP,),
        in_specs=[blk, mat, blk, mat],
        out_specs=[blk, blk],
        out_shape=[
            jax.ShapeDtypeStruct((NP, 128), jnp.float32),
            jax.ShapeDtypeStruct((NP, 128), jnp.float32),
        ],
    )(sp, Wg2, accin_p, bsum)


def _final_body(sp_ref, wg_ref, accin_ref, bsum_ref, we_ref, be_ref,
                wl0_ref, bl0_ref, wl1_ref, bl1_ref,
                wr0_ref, br0_ref, wr1_ref, br1_ref,
                emb_ref, lft_ref, rgt_ref):
    g = _l2norm_p(jnp.tanh(
        jnp.dot(sp_ref[...], wg_ref[...], preferred_element_type=jnp.float32)),
        bsum_ref[...])
    acc = accin_ref[...] + g
    emb_ref[...] = _l2norm_p(
        jnp.dot(acc, we_ref[...], preferred_element_type=jnp.float32)
        + be_ref[...], bsum_ref[...])
    lo = jnp.tanh(jnp.dot(g, wl0_ref[...], preferred_element_type=jnp.float32)
                  + bl0_ref[...]) + g
    lft_ref[...] = jnp.tanh(
        jnp.dot(lo, wl1_ref[...], preferred_element_type=jnp.float32)
        + bl1_ref[...])
    ro = jnp.tanh(jnp.dot(g, wr0_ref[...], preferred_element_type=jnp.float32)
                  + br0_ref[...]) + g
    rgt_ref[...] = jnp.tanh(
        jnp.dot(ro, wr1_ref[...], preferred_element_type=jnp.float32)
        + br1_ref[...])


def _final(sp, Wg2, accin_p, bsum, We2, be2,
           Wl02, bl02, Wl12, bl12, Wr02, br02, Wr12, br12):
    mat = pl.BlockSpec((128, 128), lambda i: (0, 0))
    vec = pl.BlockSpec((1, 128), lambda i: (0, 0))
    blk = pl.BlockSpec((BNP, 128), lambda i: (i, 0))
    return pl.pallas_call(
        _final_body,
        grid=(NP // BNP,),
        in_specs=[
            blk, mat, blk, mat,
            mat, vec, mat, vec, mat, vec, mat, vec, mat, vec,
        ],
        out_specs=[blk, blk, blk],
        out_shape=[
            jax.ShapeDtypeStruct((NP, 128), jnp.float32),
            jax.ShapeDtypeStruct((NP, 128), jnp.float32),
            jax.ShapeDtypeStruct((NP, 128), jnp.float32),
        ],
    )(sp, Wg2, accin_p, bsum, We2, be2,
      Wl02, bl02, Wl12, bl12, Wr02, br02, Wr12, br12)


def _bd(W):
    z = jnp.zeros(W.shape, W.dtype)
    return jnp.block([[W, z], [z, W]])


def _b2(b):
    return jnp.concatenate([b, b]).reshape(1, 128)


def kernel(feat, sup_indices, sup_values, W0, b0, Wg0, Wg1, We, be,
           Wl0, bl0, Wl1, bl1, Wr0, br0, Wr1, br1):
    rows = sup_indices[0]
    cols = sup_indices[1]
    bsum = jnp.kron(jnp.eye(2, dtype=jnp.float32),
                    jnp.ones((D, D), jnp.float32))
    featp = jnp.reshape(feat, (NP, 2 * D_IN))
    hp = _mlp0(featp, _bd(W0), _b2(b0))
    s1 = _spmm_sc(hp, rows, cols, sup_values)
    gp1, accp1 = _layer(s1, _bd(Wg0), hp, bsum)
    s2 = _spmm_sc(gp1, rows, cols, sup_values)
    embp, lftp, rgtp = _final(s2, _bd(Wg1), accp1, bsum, _bd(We), _b2(be),
                              _bd(Wl0), _b2(bl0), _bd(Wl1), _b2(bl1),
                              _bd(Wr0), _b2(br0), _bd(Wr1), _b2(br1))
    return (jnp.reshape(embp, (N, D)), jnp.reshape(lftp, (N, D)),
            jnp.reshape(rgtp, (N, D)))


# speedup vs baseline: 11.3473x; 1.1264x over previous
import functools

import jax
import jax.numpy as jnp
from jax import lax
from jax.experimental import pallas as pl
from jax.experimental.pallas import tpu as pltpu
from jax.experimental.pallas import tpu_sc as plsc

N = 50000
E = 800000
D_IN = 512
D = 64
H = 32
NP = N // 2

NC = 2
NS = 16
EPT = E // NS
CH = 400
NCHUNK = EPT // CH
RPT = N // NS
BN = 2000
BNP = BN // 2


def _l2norm_p(x, bsum):
    n2 = jnp.dot(x * x, bsum, preferred_element_type=jnp.float32)
    return x * lax.rsqrt(jnp.maximum(n2, 1e-24))


def _mlp0_body(feat_ref, w0_ref, b0_ref, h_ref):
    h_ref[...] = jnp.tanh(
        jnp.dot(feat_ref[...], w0_ref[...], preferred_element_type=jnp.float32)
        + b0_ref[...]
    )


def _mlp0(feat, W0, b0):
    return pl.pallas_call(
        _mlp0_body,
        grid=(N // BN,),
        in_specs=[
            pl.BlockSpec((BN, D_IN), lambda i: (i, 0)),
            pl.BlockSpec((D_IN, D), lambda i: (0, 0)),
            pl.BlockSpec((1, D), lambda i: (0, 0)),
        ],
        out_specs=pl.BlockSpec((BN, D), lambda i: (i, 0)),
        out_shape=jax.ShapeDtypeStruct((N, D), jnp.float32),
    )(feat, W0, b0.reshape(1, D))


def _spmm_body(tbl_hbm, dst_hbm, col_hbm, val_hbm, out_hbm,
               rows_buf, col_b, dst_b, val_b, table, *sems):
    c = lax.axis_index("c")
    s = lax.axis_index("s")
    gsem = sems[0:2]
    ssem = sems[2:4]
    isem = sems[4:8]

    def zb(r, carry):
        rows_buf[0, r, pl.ds(0, 16)] = jnp.zeros((16,), jnp.float32)
        rows_buf[0, r, pl.ds(16, 16)] = jnp.zeros((16,), jnp.float32)
        return carry

    lax.fori_loop(0, CH, zb, 0)
    for i in range(RPT // CH):
        pltpu.sync_copy(rows_buf.at[0], table.at[pl.ds(s * RPT + i * CH, CH)])
    rem = RPT - (RPT // CH) * CH
    if rem:
        pltpu.sync_copy(rows_buf.at[0].at[pl.ds(0, rem)],
                        table.at[pl.ds(s * RPT + (RPT // CH) * CH, rem)])
    plsc.subcore_barrier()

    base0 = s * EPT

    def issue_idx(j, q):
        b = base0 + j * CH
        pltpu.async_copy(col_hbm.at[pl.ds(b, CH)], col_b.at[q], isem[q])
        pltpu.async_copy(dst_hbm.at[pl.ds(b, CH)], dst_b.at[q], isem[q])
        pltpu.async_copy(val_hbm.at[pl.ds(b, CH)], val_b.at[q], isem[q])

    def wait_idx(j, q):
        b = base0 + j * CH
        pltpu.make_async_copy(col_hbm.at[pl.ds(b, CH)], col_b.at[q],
                              isem[q]).wait()
        pltpu.make_async_copy(dst_hbm.at[pl.ds(b, CH)], dst_b.at[q],
                              isem[q]).wait()
        pltpu.make_async_copy(val_hbm.at[pl.ds(b, CH)], val_b.at[q],
                              isem[q]).wait()

        def xf(g, carry):
            v = col_b[q, pl.ds(g * 16, 16)]
            col_b[q, pl.ds(g * 16, 16)] = v * 2 + c
            return carry

        lax.fori_loop(0, CH // 16, xf, 0)

    def issue_gather(b, q):
        pltpu.async_copy(tbl_hbm.at[col_b.at[q]], rows_buf.at[b], gsem[b])

    def wait_gather(b, q):
        pltpu.make_async_copy(tbl_hbm.at[col_b.at[q]], rows_buf.at[b],
                              gsem[b]).wait()

    def issue_scatter(b, q):
        pltpu.async_copy(rows_buf.at[b], table.at[dst_b.at[q]],
                         ssem[b], add=True)

    def wait_scatter(b, q):
        pltpu.make_async_copy(rows_buf.at[b], table.at[dst_b.at[q]],
                              ssem[b]).wait()

    def scale(b, q):
        def grp(g, carry):
            v16 = val_b[q, pl.ds(g * 16, 16)]
            e0 = g * 16
            for i in range(16):
                v = v16[i]
                rows_buf[b, e0 + i, pl.ds(0, 16)] = \
                    rows_buf[b, e0 + i, pl.ds(0, 16)] * v
                rows_buf[b, e0 + i, pl.ds(16, 16)] = \
                    rows_buf[b, e0 + i, pl.ds(16, 16)] * v
            return carry

        lax.fori_loop(0, CH // 16, grp, 0)

    def chunk_steady(j, u, g_next=True, idx_next=True):
        b = (1 + u) % 2
        q = (1 + u) % 4
        wait_scatter(1 - b, u % 4)
        if g_next:
            wait_idx(j + 1, (2 + u) % 4)
            issue_gather(1 - b, (2 + u) % 4)
        if idx_next:
            issue_idx(j + 3, u % 4)
        wait_gather(b, q)
        scale(b, q)
        issue_scatter(b, q)

    issue_idx(0, 0)
    issue_idx(1, 1)
    issue_idx(2, 2)
    wait_idx(0, 0)
    issue_gather(0, 0)
    wait_idx(1, 1)
    issue_gather(1, 1)
    issue_idx(3, 3)
    wait_gather(0, 0)
    scale(0, 0)
    issue_scatter(0, 0)

    def steady4(g, carry):
        jb = 1 + 4 * g
        for u in range(4):
            chunk_steady(jb + u, u)
        return carry

    lax.fori_loop(0, (NCHUNK - 5) // 4, steady4, 0)

    for u in range(4):
        j = NCHUNK - 4 + u
        chunk_steady(j, u, g_next=(j + 1 <= NCHUNK - 1),
                     idx_next=(j + 3 <= NCHUNK - 1))
    wait_scatter(0, 0)

    plsc.subcore_barrier()
    lo = s * RPT
    for k in range(NC):
        @pl.when(c == k)
        def _():
            pltpu.sync_copy(table.at[pl.ds(lo, RPT)],
                            out_hbm.at[pl.ds(lo, RPT), k])


def _spmm_sc(hp, dst, col, val):
    tbl = jnp.reshape(hp, (2 * N, H))
    mesh = plsc.VectorSubcoreMesh(core_axis_name="c", subcore_axis_name="s",
                                  num_cores=NC, num_subcores=NS)
    fn = functools.partial(
        pl.kernel,
        out_type=jax.ShapeDtypeStruct((N, NC, H), jnp.float32),
        mesh=mesh,
        scratch_types=[
            pltpu.VMEM((2, CH, H), jnp.float32),
            pltpu.VMEM((4, CH), jnp.int32),
            pltpu.VMEM((4, CH), jnp.int32),
            pltpu.VMEM((4, CH), jnp.float32),
            pltpu.VMEM_SHARED((N, H), jnp.float32),
            pltpu.SemaphoreType.DMA,
            pltpu.SemaphoreType.DMA,
            pltpu.SemaphoreType.DMA,
            pltpu.SemaphoreType.DMA,
            pltpu.SemaphoreType.DMA,
            pltpu.SemaphoreType.DMA,
            pltpu.SemaphoreType.DMA,
            pltpu.SemaphoreType.DMA,
        ],
        compiler_params=pltpu.CompilerParams(use_tc_tiling_on_sc=False,
                                             needs_layout_passes=False),
    )(_spmm_body)
    out = fn(tbl, dst, col, val)
    return jnp.reshape(out, (NP, 128))


def _layer_body(sp_ref, wg_ref, accin_ref, bsum_ref, gp_ref, accp_ref):
    g = _l2norm_p(jnp.tanh(
        jnp.dot(sp_ref[...], wg_ref[...], preferred_element_type=jnp.float32)),
        bsum_ref[...])
    accp_ref[...] = accin_ref[...] + g
    gp_ref[...] = g


def _layer(sp, Wg2, accin_p, bsum):
    blk = pl.BlockSpec((BNP, 128), lambda i: (i, 0))
    mat = pl.BlockSpec((128, 128), lambda i: (0, 0))
    return pl.pallas_call(
        _layer_body,
        grid=(NP // BNP,),
        in_specs=[blk, mat, blk, mat],
        out_specs=[blk, blk],
        out_shape=[
            jax.ShapeDtypeStruct((NP, 128), jnp.float32),
            jax.ShapeDtypeStruct((NP, 128), jnp.float32),
        ],
    )(sp, Wg2, accin_p, bsum)


def _final_body(sp_ref, wg_ref, accin_ref, bsum_ref, we_ref, be_ref,
                wl0_ref, bl0_ref, wl1_ref, bl1_ref,
                wr0_ref, br0_ref, wr1_ref, br1_ref,
                emb_ref, lft_ref, rgt_ref):
    g = _l2norm_p(jnp.tanh(
        jnp.dot(sp_ref[...], wg_ref[...], preferred_element_type=jnp.float32)),
        bsum_ref[...])
    acc = accin_ref[...] + g
    emb_ref[...] = _l2norm_p(
        jnp.dot(acc, we_ref[...], preferred_element_type=jnp.float32)
        + be_ref[...], bsum_ref[...])
    lo = jnp.tanh(jnp.dot(g, wl0_ref[...], preferred_element_type=jnp.float32)
                  + bl0_ref[...]) + g
    lft_ref[...] = jnp.tanh(
        jnp.dot(lo, wl1_ref[...], preferred_element_type=jnp.float32)
        + bl1_ref[...])
    ro = jnp.tanh(jnp.dot(g, wr0_ref[...], preferred_element_type=jnp.float32)
                  + br0_ref[...]) + g
    rgt_ref[...] = jnp.tanh(
        jnp.dot(ro, wr1_ref[...], preferred_element_type=jnp.float32)
        + br1_ref[...])


def _final(sp, Wg2, accin_p, bsum, We2, be2,
           Wl02, bl02, Wl12, bl12, Wr02, br02, Wr12, br12):
    mat = pl.BlockSpec((128, 128), lambda i: (0, 0))
    vec = pl.BlockSpec((1, 128), lambda i: (0, 0))
    blk = pl.BlockSpec((BNP, 128), lambda i: (i, 0))
    return pl.pallas_call(
        _final_body,
        grid=(NP // BNP,),
        in_specs=[
            blk, mat, blk, mat,
            mat, vec, mat, vec, mat, vec, mat, vec, mat, vec,
        ],
        out_specs=[blk, blk, blk],
        out_shape=[
            jax.ShapeDtypeStruct((NP, 128), jnp.float32),
            jax.ShapeDtypeStruct((NP, 128), jnp.float32),
            jax.ShapeDtypeStruct((NP, 128), jnp.float32),
        ],
    )(sp, Wg2, accin_p, bsum, We2, be2,
      Wl02, bl02, Wl12, bl12, Wr02, br02, Wr12, br12)


def _bd(W):
    z = jnp.zeros(W.shape, W.dtype)
    return jnp.block([[W, z], [z, W]])


def _b2(b):
    return jnp.concatenate([b, b]).reshape(1, 128)


def kernel(feat, sup_indices, sup_values, W0, b0, Wg0, Wg1, We, be,
           Wl0, bl0, Wl1, bl1, Wr0, br0, Wr1, br1):
    rows = sup_indices[0]
    cols = sup_indices[1]
    bsum = jnp.kron(jnp.eye(2, dtype=jnp.float32),
                    jnp.ones((D, D), jnp.float32))
    hp = jnp.reshape(_mlp0(feat, W0, b0), (NP, 128))
    s1 = _spmm_sc(hp, rows, cols, sup_values)
    gp1, accp1 = _layer(s1, _bd(Wg0), hp, bsum)
    s2 = _spmm_sc(gp1, rows, cols, sup_values)
    embp, lftp, rgtp = _final(s2, _bd(Wg1), accp1, bsum, _bd(We), _b2(be),
                              _bd(Wl0), _b2(bl0), _bd(Wl1), _b2(bl1),
                              _bd(Wr0), _b2(br0), _bd(Wr1), _b2(br1))
    return (jnp.reshape(embp, (N, D)), jnp.reshape(lftp, (N, D)),
            jnp.reshape(rgtp, (N, D)))
